# baseline (speedup 1.0000x reference)
import jax
import jax.numpy as jnp
from jax import lax
from jax.experimental import pallas as pl
from jax.experimental.pallas import tpu as pltpu

N = 4096
BR = 128
NCH = N // BR
NB = 8


def _body(data_hbm, adj_hbm, out_ref, dbuf, abuf, acc_ref, sems):
    def start(c, s):
        pltpu.make_async_copy(
            data_hbm.at[pl.ds(c * BR, BR), :], dbuf.at[s], sems.at[s, 0]).start()
        pltpu.make_async_copy(
            adj_hbm.at[pl.ds(c * BR, BR), :], abuf.at[s], sems.at[s, 1]).start()

    for s in range(NB):
        start(s, s)

    def step(i, carry):
        for s in range(NB):
            c = i * NB + s
            pltpu.make_async_copy(
                data_hbm.at[pl.ds(0, BR), :], dbuf.at[s], sems.at[s, 0]).wait()
            pltpu.make_async_copy(
                adj_hbm.at[pl.ds(0, BR), :], abuf.at[s], sems.at[s, 1]).wait()
            acc_ref[pl.ds(c * BR, BR)] = jnp.sum(dbuf[s] * abuf[s], axis=1)

            @pl.when(c + NB < NCH)
            def _pref():
                start(c + NB, s)
        return carry

    lax.fori_loop(0, NCH // NB, step, None)

    x = acc_ref[...]
    m = jnp.max(x)
    e = jnp.exp(x - m)
    out_ref[...] = e / jnp.sum(e)


def kernel(data_input, adj_matrix):
    return pl.pallas_call(
        _body,
        in_specs=[
            pl.BlockSpec(memory_space=pl.ANY),
            pl.BlockSpec(memory_space=pl.ANY),
        ],
        out_shape=jax.ShapeDtypeStruct((N,), jnp.float32),
        scratch_shapes=[
            pltpu.VMEM((NB, BR, N), jnp.float32),
            pltpu.VMEM((NB, BR, N), jnp.float32),
            pltpu.VMEM((N,), jnp.float32),
            pltpu.SemaphoreType.DMA((NB, 2)),
        ],
    )(data_input, adj_matrix)


# speedup vs baseline: 1.0038x; 1.0038x over previous
import jax
import jax.numpy as jnp
from jax.experimental import pallas as pl
from jax.experimental.pallas import tpu as pltpu

N = 4096
BR = 256
GRID = N // BR


def _body(data_ref, adj_ref, out_ref, acc_ref):
    i = pl.program_id(0)
    acc_ref[pl.ds(i * BR, BR)] = jnp.sum(adj_ref[...] * data_ref[...], axis=1)

    @pl.when(i == GRID - 1)
    def _final():
        x = acc_ref[...]
        m = jnp.max(x)
        e = jnp.exp(x - m)
        out_ref[...] = e / jnp.sum(e)


def kernel(data_input, adj_matrix):
    return pl.pallas_call(
        _body,
        grid=(GRID,),
        in_specs=[
            pl.BlockSpec((BR, N), lambda i: (i, 0)),
            pl.BlockSpec((BR, N), lambda i: (i, 0)),
        ],
        out_specs=pl.BlockSpec((N,), lambda i: (0,)),
        out_shape=jax.ShapeDtypeStruct((N,), jnp.float32),
        scratch_shapes=[pltpu.VMEM((N,), jnp.float32)],
    )(data_input, adj_matrix)
